# baseline (speedup 1.0000x reference)
import functools

import jax
import jax.numpy as jnp
from jax import lax
from jax.experimental import pallas as pl
from jax.experimental.pallas import tpu as pltpu
from jax.experimental.pallas import tpu_sc as plsc

_EPS = 1e-05
_IDX_CHUNK = 128
_CH = 128
_LN2 = 0.6931471805599453


def _rsqrt(x):
    bits = jnp.int32(0x5F3759DF) - lax.shift_right_logical(plsc.bitcast(x, jnp.int32), 1)
    y = plsc.bitcast(bits, jnp.float32)
    for _ in range(3):
        y = y * (1.5 - 0.5 * x * y * y)
    return y


def _sqrt(x):
    return x * _rsqrt(x)


def _log(x):
    bits = plsc.bitcast(x, jnp.int32)
    e = lax.shift_right_logical(bits, 23) - 127
    m_bits = (bits & jnp.int32(0x007FFFFF)) | jnp.int32(0x3F800000)
    m = plsc.bitcast(m_bits, jnp.float32)
    big = m > 1.4142135
    m = jnp.where(big, m * 0.5, m)
    e = jnp.where(big, e + 1, e).astype(jnp.float32)
    z = (m - 1.0) / (m + 1.0)
    z2 = z * z
    p = 2.0 * z * (1.0 + z2 * (1.0 / 3.0 + z2 * (0.2 + z2 * (1.0 / 7.0 + z2 / 9.0))))
    return p + e * _LN2


def _artanh(x):
    return 0.5 * _log((1.0 + x) / (1.0 - x))


def _tanh(x):
    t = jnp.exp(-2.0 * x)
    return (1.0 - t) / (1.0 + t)


def _clip_scale(sq):
    nrm = _sqrt(sq)
    return jnp.where(nrm >= 1.0, 1.0 / (nrm - _EPS), jnp.float32(1.0))


def kernel(u_idx, r_idx, v_idx, Eh, rvh, Wu, bs, bo):
    B = u_idx.shape[0]
    D = Eh.shape[1]
    PK = 128 // D
    EhP = Eh.reshape(Eh.shape[0] // PK, PK * D)
    info = plsc.get_sparse_core_info()
    nw_ = info.num_cores * info.num_subcores
    bpw = B // nw_

    mesh = plsc.VectorSubcoreMesh(core_axis_name="c", subcore_axis_name="s")

    @functools.partial(
        pl.kernel,
        out_type=jax.ShapeDtypeStruct((B,), jnp.float32),
        mesh=mesh,
        compiler_params=pltpu.CompilerParams(
            use_tc_tiling_on_sc=True, needs_layout_passes=False),
        scratch_types=[
            pltpu.VMEM((bpw,), jnp.int32),
            pltpu.VMEM((bpw,), jnp.int32),
            pltpu.VMEM((bpw,), jnp.int32),
            pltpu.VMEM((_CH, PK * D), jnp.float32),
            pltpu.VMEM((_CH, PK * D), jnp.float32),
            pltpu.VMEM((_CH, D), jnp.float32),
            pltpu.VMEM((_CH, D), jnp.float32),
            pltpu.VMEM((bpw,), jnp.float32),
            pltpu.VMEM((bpw,), jnp.float32),
            pltpu.VMEM((bpw,), jnp.float32),
            pltpu.SemaphoreType.DMA,
            pltpu.SemaphoreType.DMA,
        ],
    )
    def k(eh, rv, wu, bs_t, bo_t, uix, rix, vix, out,
          uix_v, rix_v, vix_v, u_v, v_v, ru_v, rg_v, bsu_v, bov_v, res_v,
          sem, bsem):
        wid = lax.axis_index("s") * info.num_cores + lax.axis_index("c")
        base = wid * bpw
        pltpu.sync_copy(uix.at[pl.ds(base, bpw)], uix_v)
        pltpu.sync_copy(rix.at[pl.ds(base, bpw)], rix_v)
        pltpu.sync_copy(vix.at[pl.ds(base, bpw)], vix_v)

        bdescs = []
        for j in range(bpw // _IDX_CHUNK):
            sl = pl.ds(j * _IDX_CHUNK, _IDX_CHUNK)
            bdescs.append(pltpu.async_copy(bs_t.at[uix_v.at[sl]], bsu_v.at[sl], bsem))
            bdescs.append(pltpu.async_copy(bo_t.at[vix_v.at[sl]], bov_v.at[sl], bsem))
        for d in bdescs:
            d.wait()

        def chunk(c, _):
            coff = pl.multiple_of(c * _CH, _CH)

            def grp(g, _):
                goff = pl.multiple_of(coff + g * 16, 16)
                boff = pl.multiple_of(g * 16, 16)
                iu = uix_v[pl.ds(goff, 16)]
                iv = vix_v[pl.ds(goff, 16)]
                ir = rix_v[pl.ds(goff, 16)]
                descs = []
                for j in range(16):
                    dst = pl.ds(boff + j, 1)
                    descs.append(pltpu.async_copy(
                        eh.at[pl.ds(lax.div(iu[j], PK), 1)], u_v.at[dst], sem))
                    descs.append(pltpu.async_copy(
                        eh.at[pl.ds(lax.div(iv[j], PK), 1)], v_v.at[dst], sem))
                    descs.append(pltpu.async_copy(wu.at[pl.ds(ir[j], 1)], ru_v.at[dst], sem))
                    descs.append(pltpu.async_copy(rv.at[pl.ds(ir[j], 1)], rg_v.at[dst], sem))
                for d in descs:
                    d.wait()
                return ()

            lax.fori_loop(0, _CH // 16, grp, (), unroll=False)

            def mgrp(g, _):
                boff = pl.multiple_of(g * 16, 16)
                goff = pl.multiple_of(coff + g * 16, 16)
                rows = lax.iota(jnp.int32, 16) + boff
                ucol0 = (uix_v[pl.ds(goff, 16)] & jnp.int32(PK - 1)) * D
                vcol0 = (vix_v[pl.ds(goff, 16)] & jnp.int32(PK - 1)) * D
                zero = jnp.zeros((16,), jnp.float32)
                squ = zero
                sqv = zero
                sqrg = zero
                dvr = zero
                sqp = zero
                dpv = zero
                dprg = zero
                for d in range(D):
                    col = jnp.full((16,), d, jnp.int32)
                    ud = plsc.load_gather(u_v, [rows, ucol0 + d])
                    vd = plsc.load_gather(v_v, [rows, vcol0 + d])
                    rud = plsc.load_gather(ru_v, [rows, col])
                    rgd = plsc.load_gather(rg_v, [rows, col])
                    pd = ud * rud
                    squ = squ + ud * ud
                    sqv = sqv + vd * vd
                    sqrg = sqrg + rgd * rgd
                    dvr = dvr + vd * rgd
                    sqp = sqp + pd * pd
                    dpv = dpv + pd * vd
                    dprg = dprg + pd * rgd

                su = _clip_scale(squ)
                sv = _clip_scale(sqv)
                srg = _clip_scale(sqrg)
                squ_c = squ * su * su
                nu = jnp.clip(_sqrt(squ_c), 1e-10, 1.0 - 1e-5)
                lam = (_artanh(nu) / nu) * su
                sq_uw = lam * lam * sqp
                nww = jnp.maximum(_sqrt(sq_uw), 1e-10)
                mu = (_tanh(nww) / nww) * lam
                sq_um = mu * mu * sqp
                sqx = jnp.clip(sqv * sv * sv, 0.0, 1.0 - 1e-5)
                sqy = jnp.clip(sqrg * srg * srg, 0.0, 1.0 - 1e-5)
                dot = dvr * sv * srg
                a1 = 1.0 + 2.0 * dot + sqy
                b1 = 1.0 - sqx
                c1 = 1.0 + 2.0 * dot + sqx * sqy
                av = a1 * sv / c1
                bg = b1 * srg / c1
                sq_vm = av * av * sqv + 2.0 * av * bg * dvr + bg * bg * sqrg
                dot_umvm = mu * (av * dpv + bg * dprg)
                sum_s = _clip_scale(sq_um)
                svm_s = _clip_scale(sq_vm)
                sq_um_c = sq_um * sum_s * sum_s
                sq_vm_c = sq_vm * svm_s * svm_s
                dot_c = dot_umvm * sum_s * svm_s
                sqx2 = jnp.clip(sq_um_c, 0.0, 1.0 - 1e-5)
                sqy2 = jnp.clip(sq_vm_c, 0.0, 1.0 - 1e-5)
                dot2 = -dot_c
                a2 = 1.0 + 2.0 * dot2 + sqy2
                b2 = 1.0 - sqx2
                c2 = 1.0 + 2.0 * dot2 + sqx2 * sqy2
                sq_d = a2 * a2 * sq_um_c + 2.0 * a2 * b2 * dot2 + b2 * b2 * sq_vm_c
                dn = jnp.clip(_sqrt(sq_d) / jnp.abs(c2), 1e-10, 1.0 - 1e-5)
                at = _artanh(dn)
                sqdist = 4.0 * at * at
                res = -sqdist + bsu_v[pl.ds(goff, 16)] + bov_v[pl.ds(goff, 16)]
                res_v[pl.ds(goff, 16)] = res
                return ()

            lax.fori_loop(0, _CH // 16, mgrp, (), unroll=False)
            return ()

        lax.fori_loop(0, bpw // _CH, chunk, (), unroll=False)
        pltpu.sync_copy(res_v, out.at[pl.ds(base, bpw)])

    return k(EhP, rvh, Wu, bs, bo, u_idx, r_idx, v_idx)


# speedup vs baseline: 1.5378x; 1.5378x over previous
import functools

import jax
import jax.numpy as jnp
from jax import lax
from jax.experimental import pallas as pl
from jax.experimental.pallas import tpu as pltpu
from jax.experimental.pallas import tpu_sc as plsc

_EPS = 1e-05
_IDX_CHUNK = 128
_CH = 128
_LN2 = 0.6931471805599453


def _rsqrt(x):
    bits = jnp.int32(0x5F3759DF) - lax.shift_right_logical(plsc.bitcast(x, jnp.int32), 1)
    y = plsc.bitcast(bits, jnp.float32)
    for _ in range(3):
        y = y * (1.5 - 0.5 * x * y * y)
    return y


def _sqrt(x):
    return x * _rsqrt(x)


def _log(x):
    bits = plsc.bitcast(x, jnp.int32)
    e = lax.shift_right_logical(bits, 23) - 127
    m_bits = (bits & jnp.int32(0x007FFFFF)) | jnp.int32(0x3F800000)
    m = plsc.bitcast(m_bits, jnp.float32)
    big = m > 1.4142135
    m = jnp.where(big, m * 0.5, m)
    e = jnp.where(big, e + 1, e).astype(jnp.float32)
    z = (m - 1.0) / (m + 1.0)
    z2 = z * z
    p = 2.0 * z * (1.0 + z2 * (1.0 / 3.0 + z2 * (0.2 + z2 * (1.0 / 7.0 + z2 / 9.0))))
    return p + e * _LN2


def _artanh(x):
    return 0.5 * _log((1.0 + x) / (1.0 - x))


def _tanh(x):
    t = jnp.exp(-2.0 * x)
    return (1.0 - t) / (1.0 + t)


def _clip_scale(sq):
    nrm = _sqrt(sq)
    return jnp.where(nrm >= 1.0, 1.0 / (nrm - _EPS), jnp.float32(1.0))


def kernel(u_idx, r_idx, v_idx, Eh, rvh, Wu, bs, bo):
    B = u_idx.shape[0]
    D = Eh.shape[1]
    info = plsc.get_sparse_core_info()
    nw_ = info.num_cores * info.num_subcores
    bpw = B // nw_

    mesh = plsc.VectorSubcoreMesh(core_axis_name="c", subcore_axis_name="s")

    @functools.partial(
        pl.kernel,
        out_type=jax.ShapeDtypeStruct((B,), jnp.float32),
        mesh=mesh,
        compiler_params=pltpu.CompilerParams(
            use_tc_tiling_on_sc=True, needs_layout_passes=False),
        scratch_types=[
            pltpu.VMEM((bpw,), jnp.int32),
            pltpu.VMEM((bpw,), jnp.int32),
            pltpu.VMEM((bpw,), jnp.int32),
            pltpu.VMEM((_CH, D), jnp.float32),
            pltpu.VMEM((_CH, D), jnp.float32),
            pltpu.VMEM((_CH, D), jnp.float32),
            pltpu.VMEM((_CH, D), jnp.float32),
            pltpu.VMEM((bpw,), jnp.float32),
            pltpu.VMEM((bpw,), jnp.float32),
            pltpu.VMEM((bpw,), jnp.float32),
            pltpu.SemaphoreType.DMA,
            pltpu.SemaphoreType.DMA,
        ],
    )
    def k(eh, rv, wu, bs_t, bo_t, uix, rix, vix, out,
          uix_v, rix_v, vix_v, u_v, v_v, ru_v, rg_v, bsu_v, bov_v, res_v,
          sem, bsem):
        wid = lax.axis_index("s") * info.num_cores + lax.axis_index("c")
        base = wid * bpw
        pltpu.sync_copy(uix.at[pl.ds(base, bpw)], uix_v)
        pltpu.sync_copy(rix.at[pl.ds(base, bpw)], rix_v)
        pltpu.sync_copy(vix.at[pl.ds(base, bpw)], vix_v)

        bdescs = []
        for j in range(bpw // _IDX_CHUNK):
            sl = pl.ds(j * _IDX_CHUNK, _IDX_CHUNK)
            bdescs.append(pltpu.async_copy(bs_t.at[uix_v.at[sl]], bsu_v.at[sl], bsem))
            bdescs.append(pltpu.async_copy(bo_t.at[vix_v.at[sl]], bov_v.at[sl], bsem))
        for d in bdescs:
            d.wait()

        def chunk(c, _):
            coff = pl.multiple_of(c * _CH, _CH)

            def grp(g, _):
                goff = pl.multiple_of(coff + g * 16, 16)
                boff = pl.multiple_of(g * 16, 16)
                iu = uix_v[pl.ds(goff, 16)]
                iv = vix_v[pl.ds(goff, 16)]
                ir = rix_v[pl.ds(goff, 16)]
                descs = []
                for j in range(16):
                    dst = pl.ds(boff + j, 1)
                    descs.append(pltpu.async_copy(eh.at[pl.ds(iu[j], 1)], u_v.at[dst], sem))
                    descs.append(pltpu.async_copy(eh.at[pl.ds(iv[j], 1)], v_v.at[dst], sem))
                    descs.append(pltpu.async_copy(wu.at[pl.ds(ir[j], 1)], ru_v.at[dst], sem))
                    descs.append(pltpu.async_copy(rv.at[pl.ds(ir[j], 1)], rg_v.at[dst], sem))
                for d in descs:
                    d.wait()
                return ()

            lax.fori_loop(0, _CH // 16, grp, (), unroll=False)

            def mgrp(g, _):
                boff = pl.multiple_of(g * 16, 16)
                goff = pl.multiple_of(coff + g * 16, 16)
                rows = lax.iota(jnp.int32, 16) + boff
                zero = jnp.zeros((16,), jnp.float32)
                squ = zero
                sqv = zero
                sqrg = zero
                dvr = zero
                sqp = zero
                dpv = zero
                dprg = zero
                for d in range(D):
                    col = jnp.full((16,), d, jnp.int32)
                    ud = plsc.load_gather(u_v, [rows, col])
                    vd = plsc.load_gather(v_v, [rows, col])
                    rud = plsc.load_gather(ru_v, [rows, col])
                    rgd = plsc.load_gather(rg_v, [rows, col])
                    pd = ud * rud
                    squ = squ + ud * ud
                    sqv = sqv + vd * vd
                    sqrg = sqrg + rgd * rgd
                    dvr = dvr + vd * rgd
                    sqp = sqp + pd * pd
                    dpv = dpv + pd * vd
                    dprg = dprg + pd * rgd

                su = _clip_scale(squ)
                sv = _clip_scale(sqv)
                srg = _clip_scale(sqrg)
                squ_c = squ * su * su
                nu = jnp.clip(_sqrt(squ_c), 1e-10, 1.0 - 1e-5)
                lam = (_artanh(nu) / nu) * su
                sq_uw = lam * lam * sqp
                nww = jnp.maximum(_sqrt(sq_uw), 1e-10)
                mu = (_tanh(nww) / nww) * lam
                sq_um = mu * mu * sqp
                sqx = jnp.clip(sqv * sv * sv, 0.0, 1.0 - 1e-5)
                sqy = jnp.clip(sqrg * srg * srg, 0.0, 1.0 - 1e-5)
                dot = dvr * sv * srg
                a1 = 1.0 + 2.0 * dot + sqy
                b1 = 1.0 - sqx
                c1 = 1.0 + 2.0 * dot + sqx * sqy
                av = a1 * sv / c1
                bg = b1 * srg / c1
                sq_vm = av * av * sqv + 2.0 * av * bg * dvr + bg * bg * sqrg
                dot_umvm = mu * (av * dpv + bg * dprg)
                sum_s = _clip_scale(sq_um)
                svm_s = _clip_scale(sq_vm)
                sq_um_c = sq_um * sum_s * sum_s
                sq_vm_c = sq_vm * svm_s * svm_s
                dot_c = dot_umvm * sum_s * svm_s
                sqx2 = jnp.clip(sq_um_c, 0.0, 1.0 - 1e-5)
                sqy2 = jnp.clip(sq_vm_c, 0.0, 1.0 - 1e-5)
                dot2 = -dot_c
                a2 = 1.0 + 2.0 * dot2 + sqy2
                b2 = 1.0 - sqx2
                c2 = 1.0 + 2.0 * dot2 + sqx2 * sqy2
                sq_d = a2 * a2 * sq_um_c + 2.0 * a2 * b2 * dot2 + b2 * b2 * sq_vm_c
                dn = jnp.clip(_sqrt(sq_d) / jnp.abs(c2), 1e-10, 1.0 - 1e-5)
                at = _artanh(dn)
                sqdist = 4.0 * at * at
                res = -sqdist + bsu_v[pl.ds(goff, 16)] + bov_v[pl.ds(goff, 16)]
                res_v[pl.ds(goff, 16)] = res
                return ()

            lax.fori_loop(0, _CH // 16, mgrp, (), unroll=False)
            return ()

        lax.fori_loop(0, bpw // _CH, chunk, (), unroll=False)
        pltpu.sync_copy(res_v, out.at[pl.ds(base, bpw)])

    return k(Eh, rvh, Wu, bs, bo, u_idx, r_idx, v_idx)
